# baseline (speedup 1.0000x reference)
import jax
import jax.numpy as jnp
from jax.experimental import pallas as pl

DIM = 1024
HEADS = 16
HD = DIM // HEADS
NEXP = 8
DSTEP = DIM // NEXP
SCALE = HD ** -0.5


def _qkv_kernel(x_ref, em_ref, w_ref, o_ref):
    x = x_ref[...]
    em = em_ref[0, 0]
    d_tok = (em + 1) * DSTEP
    col = jax.lax.broadcasted_iota(jnp.int32, x.shape, 1)
    xm = jnp.where(col < d_tok[:, None], x, 0.0)
    o_ref[...] = jnp.dot(xm.astype(jnp.bfloat16), w_ref[...].astype(jnp.bfloat16),
                         preferred_element_type=jnp.float32)


def _attn_kernel(q_ref, k_ref, v_ref, o_ref):
    q = q_ref[0, 0]
    k = k_ref[0, 0]
    v = v_ref[0, 0]
    s = jax.lax.dot_general(q.astype(jnp.bfloat16), k.astype(jnp.bfloat16),
                            (((1,), (1,)), ((), ())),
                            preferred_element_type=jnp.float32) * SCALE
    m = jnp.max(s, axis=-1, keepdims=True)
    p = jnp.exp(s - m)
    l = jnp.sum(p, axis=-1, keepdims=True)
    o = jnp.dot(p.astype(jnp.bfloat16), v.astype(jnp.bfloat16),
                preferred_element_type=jnp.float32) / l
    o_ref[0, 0] = o


def _proj_kernel(x_ref, em_ref, w_ref, b_ref, o_ref):
    x = x_ref[...]
    y = jnp.dot(x.astype(jnp.bfloat16), w_ref[...].astype(jnp.bfloat16),
                preferred_element_type=jnp.float32)
    y = y + b_ref[...][None, :]
    em = em_ref[0, 0]
    d_tok = (em + 1) * DSTEP
    col = jax.lax.broadcasted_iota(jnp.int32, y.shape, 1)
    o_ref[...] = jnp.where(col < d_tok[:, None], y, 0.0)


def kernel(input_tokens, expert_mask, Wqkv, Wproj, bproj):
    B, N, D = input_tokens.shape
    R = 512
    BQ = 512
    nrow = (B * N) // R

    x2 = input_tokens.reshape(B * N, D)
    em3 = expert_mask.reshape(nrow, 1, R)
    wqkv_t = Wqkv.T
    wproj_t = Wproj.T

    qkv = pl.pallas_call(
        _qkv_kernel,
        grid=(nrow,),
        in_specs=[
            pl.BlockSpec((R, D), lambda i: (i, 0)),
            pl.BlockSpec((1, 1, R), lambda i: (i, 0, 0)),
            pl.BlockSpec((D, 3 * D), lambda i: (0, 0)),
        ],
        out_specs=pl.BlockSpec((R, 3 * D), lambda i: (i, 0)),
        out_shape=jax.ShapeDtypeStruct((B * N, 3 * D), jnp.float32),
    )(x2, em3, wqkv_t)

    qkv4 = qkv.reshape(B, N, 3 * HEADS, HD).transpose(0, 2, 1, 3)
    x_attn = pl.pallas_call(
        _attn_kernel,
        grid=(B, HEADS, N // BQ),
        in_specs=[
            pl.BlockSpec((1, 1, BQ, HD), lambda b, h, qi: (b, h, qi, 0)),
            pl.BlockSpec((1, 1, N, HD), lambda b, h, qi: (b, HEADS + h, 0, 0)),
            pl.BlockSpec((1, 1, N, HD), lambda b, h, qi: (b, 2 * HEADS + h, 0, 0)),
        ],
        out_specs=pl.BlockSpec((1, 1, BQ, HD), lambda b, h, qi: (b, h, qi, 0)),
        out_shape=jax.ShapeDtypeStruct((B, HEADS, N, HD), jnp.float32),
    )(qkv4, qkv4, qkv4)
    x_attn = x_attn.transpose(0, 2, 1, 3)

    y = pl.pallas_call(
        _proj_kernel,
        grid=(nrow,),
        in_specs=[
            pl.BlockSpec((R, D), lambda i: (i, 0)),
            pl.BlockSpec((1, 1, R), lambda i: (i, 0, 0)),
            pl.BlockSpec((D, D), lambda i: (0, 0)),
            pl.BlockSpec((D,), lambda i: (0,)),
        ],
        out_specs=pl.BlockSpec((R, D), lambda i: (i, 0)),
        out_shape=jax.ShapeDtypeStruct((B * N, D), jnp.float32),
    )(x_attn.reshape(B * N, D), em3, wproj_t, bproj)

    return y.reshape(B, N, D)


# speedup vs baseline: 1.4336x; 1.4336x over previous
import functools

import jax
import jax.numpy as jnp
from jax.experimental import pallas as pl

DIM = 1024
HEADS = 16
HD = DIM // HEADS
NEXP = 8
DSTEP = DIM // NEXP
SCALE = HD ** -0.5


def _qkv_kernel(x_ref, em_ref, w_ref, o_ref):
    x = x_ref[...]
    em = em_ref[0, 0]
    d_tok = (em + 1) * DSTEP
    col = jax.lax.broadcasted_iota(jnp.int32, x.shape, 1)
    xm = jnp.where(col < d_tok[:, None], x, 0.0).astype(jnp.bfloat16)
    acc = jnp.dot(xm, w_ref[...], preferred_element_type=jnp.float32)
    o_ref[...] = acc.astype(jnp.bfloat16)


def _attn_kernel(q_ref, k_ref, v_ref, o_ref, *, bq, chunk):
    k = k_ref[0, 0]
    v = v_ref[0, 0]
    for c in range(bq // chunk):
        q = q_ref[0, 0, c * chunk:(c + 1) * chunk]
        s = jax.lax.dot_general(q, k, (((1,), (1,)), ((), ())),
                                preferred_element_type=jnp.float32) * SCALE
        m = jnp.max(s, axis=-1, keepdims=True)
        p = jnp.exp(s - m)
        l = jnp.sum(p, axis=-1, keepdims=True)
        o = jnp.dot(p.astype(jnp.bfloat16), v,
                    preferred_element_type=jnp.float32) / l
        o_ref[0, 0, c * chunk:(c + 1) * chunk] = o.astype(jnp.bfloat16)


def _proj_kernel(x_ref, em_ref, w_ref, b_ref, o_ref):
    x = x_ref[...]
    y = jnp.dot(x, w_ref[...], preferred_element_type=jnp.float32)
    y = y + b_ref[...][None, :]
    em = em_ref[0, 0]
    d_tok = (em + 1) * DSTEP
    col = jax.lax.broadcasted_iota(jnp.int32, y.shape, 1)
    o_ref[...] = jnp.where(col < d_tok[:, None], y, 0.0)


def kernel(input_tokens, expert_mask, Wqkv, Wproj, bproj):
    B, N, D = input_tokens.shape
    R = 512
    BQ = 1024
    CHUNK = 256
    nrow = (B * N) // R

    x2 = input_tokens.reshape(B * N, D)
    em3 = expert_mask.reshape(nrow, 1, R)
    wqkv_t = Wqkv.T.astype(jnp.bfloat16)
    wproj_t = Wproj.T.astype(jnp.bfloat16)

    qkv = pl.pallas_call(
        _qkv_kernel,
        grid=(nrow,),
        in_specs=[
            pl.BlockSpec((R, D), lambda i: (i, 0)),
            pl.BlockSpec((1, 1, R), lambda i: (i, 0, 0)),
            pl.BlockSpec((D, 3 * D), lambda i: (0, 0)),
        ],
        out_specs=pl.BlockSpec((R, 3 * D), lambda i: (i, 0)),
        out_shape=jax.ShapeDtypeStruct((B * N, 3 * D), jnp.bfloat16),
    )(x2, em3, wqkv_t)

    qkv4 = qkv.reshape(B, N, 3 * HEADS, HD).transpose(0, 2, 1, 3)
    attn_body = functools.partial(_attn_kernel, bq=BQ, chunk=CHUNK)
    x_attn = pl.pallas_call(
        attn_body,
        grid=(B, HEADS, N // BQ),
        in_specs=[
            pl.BlockSpec((1, 1, BQ, HD), lambda b, h, qi: (b, h, qi, 0)),
            pl.BlockSpec((1, 1, N, HD), lambda b, h, qi: (b, HEADS + h, 0, 0)),
            pl.BlockSpec((1, 1, N, HD), lambda b, h, qi: (b, 2 * HEADS + h, 0, 0)),
        ],
        out_specs=pl.BlockSpec((1, 1, BQ, HD), lambda b, h, qi: (b, h, qi, 0)),
        out_shape=jax.ShapeDtypeStruct((B, HEADS, N, HD), jnp.bfloat16),
    )(qkv4, qkv4, qkv4)
    x_attn = x_attn.transpose(0, 2, 1, 3)

    y = pl.pallas_call(
        _proj_kernel,
        grid=(nrow,),
        in_specs=[
            pl.BlockSpec((R, D), lambda i: (i, 0)),
            pl.BlockSpec((1, 1, R), lambda i: (i, 0, 0)),
            pl.BlockSpec((D, D), lambda i: (0, 0)),
            pl.BlockSpec((D,), lambda i: (0,)),
        ],
        out_specs=pl.BlockSpec((R, D), lambda i: (i, 0)),
        out_shape=jax.ShapeDtypeStruct((B * N, D), jnp.float32),
    )(x_attn.reshape(B * N, D), em3, wproj_t, bproj)

    return y.reshape(B, N, D)
